# baseline (speedup 1.0000x reference)
import functools

import jax
import jax.numpy as jnp
from jax.experimental import pallas as pl
from jax.experimental.pallas import tpu as pltpu

_NEG_RATIO = 3


def _rowsum(x, ones_col):
    return jax.lax.dot_general(x, ones_col, (((1,), (0,)), ((), ())),
                               preferred_element_type=jnp.float32)


def _mbl_kernel(lab_ref, lt_ref, ploc_ref, gloc_ref, out_loc_ref, out_cls_ref,
                losses_s, np_s, acc_s, *, B, A, C):
    b = pl.program_id(0)
    ones_col = jnp.ones((A, 1), jnp.float32)
    ones_row = jnp.ones((1, C), jnp.float32)

    lab = lab_ref[0]
    lt = lt_ref[0]
    posf = (lab > 0).astype(jnp.float32)

    m = jnp.max(lt, axis=0, keepdims=True)
    e = jnp.exp(lt - m)
    se = jax.lax.dot_general(ones_row, e, (((1,), (0,)), ((), ())),
                             preferred_element_type=jnp.float32)
    lse = jnp.log(se) + m
    cls_iota = jax.lax.broadcasted_iota(jnp.int32, (C, A), 0)
    oh = jnp.where(cls_iota == lab, lt, 0.0)
    picked = jax.lax.dot_general(ones_row, oh, (((1,), (0,)), ((), ())),
                                 preferred_element_type=jnp.float32)
    ce = lse - picked

    losses_s[pl.ds(b, 1), :] = ce * (1.0 - posf)

    numpos = _rowsum(posf, ones_col)
    posce = _rowsum(ce * posf, ones_col)

    d = ploc_ref[0] - gloc_ref[0]
    ad = jnp.abs(d)
    sl1 = jnp.where(ad < 1.0, 0.5 * d * d, ad - 0.5)
    colsum = jnp.sum(sl1, axis=0, keepdims=True)
    locl = _rowsum(colsum * posf, ones_col)

    np_s[pl.ds(b, 1), :] = jnp.broadcast_to(numpos, (1, 128))

    @pl.when(b == 0)
    def _init():
        acc_s[0] = 0.0
        acc_s[1] = 0.0

    acc_s[0] = acc_s[0] + locl[0, 0]
    acc_s[1] = acc_s[1] + posce[0, 0]

    @pl.when(b == B - 1)
    def _finish():
        allb = losses_s[:, :]
        bits = jax.lax.bitcast_convert_type(allb, jnp.int32)
        npvec = np_s[:, 0:1]
        k = jnp.minimum(_NEG_RATIO * npvec, float(A - 1))

        lo = jnp.zeros((B, 1), jnp.int32)
        hi = jnp.max(bits, axis=1, keepdims=True)

        def body(_, carry):
            lo, hi = carry
            mid = lo + (hi - lo + 1) // 2
            mask = (bits >= mid).astype(jnp.float32)
            cnt = _rowsum(mask, ones_col)
            ge = cnt >= k
            return jnp.where(ge, mid, lo), jnp.where(ge, hi, mid - 1)

        lo, hi = jax.lax.fori_loop(0, 31, body, (lo, hi))
        tv = jax.lax.bitcast_convert_type(lo, jnp.float32)
        gtf = (bits > lo).astype(jnp.float32)
        cnt_gt = _rowsum(gtf, ones_col)
        sum_gt = _rowsum(allb * gtf, ones_col)
        topk = sum_gt + (k - cnt_gt) * tv
        topk = jnp.where(k >= 1.0, topk, 0.0)

        n = jnp.sum(npvec)
        out_loc_ref[:, :] = jnp.reshape(acc_s[0] / n, (1, 1))
        out_cls_ref[:, :] = jnp.reshape((acc_s[1] + jnp.sum(topk)) / n, (1, 1))


def kernel(pred_loc, pred_label, gt_loc, gt_label):
    B, A, C = pred_label.shape
    labT = gt_label.reshape(B, 1, A)
    ltT = pred_label.transpose(0, 2, 1)
    plocT = pred_loc.transpose(0, 2, 1)
    glocT = gt_loc.transpose(0, 2, 1)

    grid = (B,)
    out_loc, out_cls = pl.pallas_call(
        functools.partial(_mbl_kernel, B=B, A=A, C=C),
        grid=grid,
        in_specs=[
            pl.BlockSpec((1, 1, A), lambda b: (b, 0, 0)),
            pl.BlockSpec((1, C, A), lambda b: (b, 0, 0)),
            pl.BlockSpec((1, 4, A), lambda b: (b, 0, 0)),
            pl.BlockSpec((1, 4, A), lambda b: (b, 0, 0)),
        ],
        out_specs=[
            pl.BlockSpec((1, 1), lambda b: (0, 0)),
            pl.BlockSpec((1, 1), lambda b: (0, 0)),
        ],
        out_shape=[
            jax.ShapeDtypeStruct((1, 1), jnp.float32),
            jax.ShapeDtypeStruct((1, 1), jnp.float32),
        ],
        scratch_shapes=[
            pltpu.VMEM((B, A), jnp.float32),
            pltpu.VMEM((B, 128), jnp.float32),
            pltpu.SMEM((2,), jnp.float32),
        ],
    )(labT, ltT, plocT, glocT)
    return (out_loc.reshape(()), out_cls.reshape(()))


# speedup vs baseline: 1.3290x; 1.3290x over previous
import functools

import jax
import jax.numpy as jnp
from jax.experimental import pallas as pl
from jax.experimental.pallas import tpu as pltpu

_NEG_RATIO = 3


def _mbl_kernel(lab_ref, lt_ref, ploc_ref, gloc_ref, out_loc_ref, out_cls_ref,
                losses_s, np_s, acc_s, *, B, A, C):
    b = pl.program_id(0)

    lab = lab_ref[0]
    lt = lt_ref[0]
    pos = lab > 0

    m = jnp.max(lt, axis=0, keepdims=True)
    se = jnp.sum(jnp.exp(lt - m), axis=0, keepdims=True)
    lse = jnp.log(se) + m
    cls_iota = jax.lax.broadcasted_iota(jnp.int32, (C, A), 0)
    picked = jnp.sum(jnp.where(cls_iota == lab, lt, 0.0), axis=0, keepdims=True)
    ce = lse - picked

    posf = pos.astype(jnp.float32)
    losses = ce - ce * posf
    losses_s[pl.ds(b, 1), :] = losses

    d = ploc_ref[0] - gloc_ref[0]
    ad = jnp.abs(d)
    sl1 = jnp.where(ad < 1.0, 0.5 * d * d, ad - 0.5)
    cols = jnp.sum(sl1, axis=0, keepdims=True)

    red = jnp.concatenate([posf, ce * posf, cols * posf], axis=0)
    s3 = jnp.sum(red, axis=1, keepdims=True)
    numpos = s3[0, 0].astype(jnp.int32)
    posce = s3[1, 0]
    locl = s3[2, 0]

    np_s[pl.ds(b, 1), :] = jnp.broadcast_to(numpos, (1, 128))

    @pl.when(b == 0)
    def _init():
        acc_s[0] = 0.0
        acc_s[1] = 0.0

    acc_s[0] = acc_s[0] + locl
    acc_s[1] = acc_s[1] + posce

    @pl.when(b == B - 1)
    def _finish():
        allb = losses_s[:, :]
        bits = jax.lax.bitcast_convert_type(allb, jnp.int32)
        npvec = np_s[:, 0:1]
        k = jnp.minimum(_NEG_RATIO * npvec, A - 1)

        lo = jnp.zeros((B, 1), jnp.int32)
        hi = jnp.max(bits, axis=1, keepdims=True)

        def body(_, carry):
            lo, hi = carry
            mid = lo + (hi - lo + 1) // 2
            cnt = jnp.sum((bits >= mid).astype(jnp.int32), axis=1,
                          keepdims=True)
            ge = cnt >= k
            return jnp.where(ge, mid, lo), jnp.where(ge, hi, mid - 1)

        lo, hi = jax.lax.fori_loop(0, 31, body, (lo, hi))
        tv = jax.lax.bitcast_convert_type(lo, jnp.float32)
        gt = bits > lo
        cnt_gt = jnp.sum(gt.astype(jnp.int32), axis=1, keepdims=True)
        sum_gt = jnp.sum(jnp.where(gt, allb, 0.0), axis=1, keepdims=True)
        topk = sum_gt + (k - cnt_gt).astype(jnp.float32) * tv
        topk = jnp.where(k >= 1, topk, 0.0)

        n = jnp.sum(npvec).astype(jnp.float32)
        out_loc_ref[:, :] = jnp.reshape(acc_s[0] / n, (1, 1))
        out_cls_ref[:, :] = jnp.reshape((acc_s[1] + jnp.sum(topk)) / n, (1, 1))


def kernel(pred_loc, pred_label, gt_loc, gt_label):
    B, A, C = pred_label.shape
    labT = gt_label.reshape(B, 1, A)
    ltT = pred_label.transpose(0, 2, 1)
    plocT = pred_loc.transpose(0, 2, 1)
    glocT = gt_loc.transpose(0, 2, 1)

    grid = (B,)
    out_loc, out_cls = pl.pallas_call(
        functools.partial(_mbl_kernel, B=B, A=A, C=C),
        grid=grid,
        in_specs=[
            pl.BlockSpec((1, 1, A), lambda b: (b, 0, 0)),
            pl.BlockSpec((1, C, A), lambda b: (b, 0, 0)),
            pl.BlockSpec((1, 4, A), lambda b: (b, 0, 0)),
            pl.BlockSpec((1, 4, A), lambda b: (b, 0, 0)),
        ],
        out_specs=[
            pl.BlockSpec((1, 1), lambda b: (0, 0)),
            pl.BlockSpec((1, 1), lambda b: (0, 0)),
        ],
        out_shape=[
            jax.ShapeDtypeStruct((1, 1), jnp.float32),
            jax.ShapeDtypeStruct((1, 1), jnp.float32),
        ],
        scratch_shapes=[
            pltpu.VMEM((B, A), jnp.float32),
            pltpu.VMEM((B, 128), jnp.int32),
            pltpu.SMEM((2,), jnp.float32),
        ],
    )(labT, ltT, plocT, glocT)
    return (out_loc.reshape(()), out_cls.reshape(()))


# speedup vs baseline: 1.4717x; 1.1074x over previous
import functools

import jax
import jax.numpy as jnp
from jax.experimental import pallas as pl
from jax.experimental.pallas import tpu as pltpu

_NEG_RATIO = 3


def _mbl_kernel(lab_ref, lt_ref, ploc_ref, gloc_ref, out_loc_ref, out_cls_ref,
                losses_s, np_s, acc_s, *, B, A, C):
    b = pl.program_id(0)

    lab = lab_ref[0]
    lt = lt_ref[0]
    pos = lab > 0

    m = jnp.max(lt, axis=0, keepdims=True)
    se = jnp.sum(jnp.exp(lt - m), axis=0, keepdims=True)
    lse = jnp.log(se) + m
    cls_iota = jax.lax.broadcasted_iota(jnp.int32, (C, A), 0)
    picked = jnp.sum(jnp.where(cls_iota == lab, lt, 0.0), axis=0, keepdims=True)
    ce = lse - picked

    losses = jnp.where(pos, 0.0, ce)
    losses_s[pl.ds(b, 1), :] = losses

    numpos = jnp.sum(pos.astype(jnp.int32))
    posce = jnp.sum(jnp.where(pos, ce, 0.0))

    d = ploc_ref[0] - gloc_ref[0]
    ad = jnp.abs(d)
    sl1 = jnp.where(ad < 1.0, 0.5 * d * d, ad - 0.5)
    locl = jnp.sum(jnp.where(pos, sl1, 0.0))

    np_s[pl.ds(b, 1), :] = jnp.broadcast_to(numpos, (1, 128))

    @pl.when(b == 0)
    def _init():
        acc_s[0] = 0.0
        acc_s[1] = 0.0

    acc_s[0] = acc_s[0] + locl
    acc_s[1] = acc_s[1] + posce

    @pl.when(b == B - 1)
    def _finish():
        allb = losses_s[:, :]
        tb = jax.lax.shift_right_logical(
            jax.lax.bitcast_convert_type(allb, jnp.int32), 13)
        npvec = np_s[:, 0:1]
        k = jnp.minimum(_NEG_RATIO * npvec, A - 1)

        lo = jnp.zeros((B, 1), jnp.int32)
        hi = jnp.max(tb, axis=1, keepdims=True)

        def body(_, carry):
            lo, hi = carry
            mid = lo + (hi - lo + 1) // 2
            cnt = jnp.sum((tb >= mid).astype(jnp.int32), axis=1,
                          keepdims=True)
            ge = cnt >= k
            return jnp.where(ge, mid, lo), jnp.where(ge, hi, mid - 1)

        lo, hi = jax.lax.fori_loop(0, 19, body, (lo, hi))
        tv = jax.lax.bitcast_convert_type(
            jax.lax.shift_left(lo, 13), jnp.float32)
        gt = tb > lo
        cnt_gt = jnp.sum(gt.astype(jnp.int32), axis=1, keepdims=True)
        sum_gt = jnp.sum(jnp.where(gt, allb, 0.0), axis=1, keepdims=True)
        topk = sum_gt + (k - cnt_gt).astype(jnp.float32) * tv
        topk = jnp.where(k >= 1, topk, 0.0)

        n = jnp.sum(npvec).astype(jnp.float32)
        out_loc_ref[:, :] = jnp.reshape(acc_s[0] / n, (1, 1))
        out_cls_ref[:, :] = jnp.reshape((acc_s[1] + jnp.sum(topk)) / n, (1, 1))


def kernel(pred_loc, pred_label, gt_loc, gt_label):
    B, A, C = pred_label.shape
    labT = gt_label.reshape(B, 1, A)
    ltT = pred_label.transpose(0, 2, 1)
    plocT = pred_loc.transpose(0, 2, 1)
    glocT = gt_loc.transpose(0, 2, 1)

    grid = (B,)
    out_loc, out_cls = pl.pallas_call(
        functools.partial(_mbl_kernel, B=B, A=A, C=C),
        grid=grid,
        in_specs=[
            pl.BlockSpec((1, 1, A), lambda b: (b, 0, 0)),
            pl.BlockSpec((1, C, A), lambda b: (b, 0, 0)),
            pl.BlockSpec((1, 4, A), lambda b: (b, 0, 0)),
            pl.BlockSpec((1, 4, A), lambda b: (b, 0, 0)),
        ],
        out_specs=[
            pl.BlockSpec((1, 1), lambda b: (0, 0)),
            pl.BlockSpec((1, 1), lambda b: (0, 0)),
        ],
        out_shape=[
            jax.ShapeDtypeStruct((1, 1), jnp.float32),
            jax.ShapeDtypeStruct((1, 1), jnp.float32),
        ],
        scratch_shapes=[
            pltpu.VMEM((B, A), jnp.float32),
            pltpu.VMEM((B, 128), jnp.int32),
            pltpu.SMEM((2,), jnp.float32),
        ],
    )(labT, ltT, plocT, glocT)
    return (out_loc.reshape(()), out_cls.reshape(()))
